# baseline (speedup 1.0000x reference)
import functools

import jax
import jax.numpy as jnp
from jax import lax
from jax.experimental import pallas as pl
from jax.experimental.pallas import tpu as pltpu
from jax.experimental.pallas import tpu_sc as plsc

B = 16384
D = 32
NC = 2
NS = 16
NW = NC * NS
BPW = B // NW
CHUNK = 128
NCHUNK = BPW // CHUNK


@functools.partial(
    pl.kernel,
    out_type=jax.ShapeDtypeStruct((B,), jnp.float32),
    mesh=plsc.VectorSubcoreMesh(core_axis_name="c", subcore_axis_name="s"),
    compiler_params=pltpu.CompilerParams(use_tc_tiling_on_sc=False),
    scratch_types=[
        pltpu.VMEM((NCHUNK, CHUNK), jnp.int32),
        pltpu.VMEM((NCHUNK, CHUNK), jnp.int32),
        pltpu.VMEM((D, BPW), jnp.float32),
        pltpu.VMEM((D, BPW), jnp.float32),
        pltpu.VMEM((BPW,), jnp.float32),
        pltpu.VMEM((BPW,), jnp.float32),
        pltpu.VMEM((16,), jnp.float32),
        pltpu.VMEM((BPW,), jnp.float32),
        pltpu.SemaphoreType.DMA,
    ],
)
def _mf_sc(uid_hbm, iid_hbm, uemb_t, iemb_t, ub_t, ib_t, gb_hbm,
           out_hbm, uid_v, iid_v, ur_t, ir_t, ubr, ibr, gbv, outv, sem):
    wid = lax.axis_index("s") * NC + lax.axis_index("c")
    base = wid * BPW

    for j in range(NCHUNK):
        row = wid * NCHUNK + j
        pltpu.sync_copy(uid_hbm.at[row], uid_v.at[j])
        pltpu.sync_copy(iid_hbm.at[row], iid_v.at[j])
    pltpu.sync_copy(gb_hbm, gbv)

    copies = []
    for j in range(NCHUNK):
        sl = pl.ds(j * CHUNK, CHUNK)
        copies.append(pltpu.async_copy(ub_t.at[0].at[uid_v.at[j]], ubr.at[sl], sem))
        copies.append(pltpu.async_copy(ib_t.at[0].at[iid_v.at[j]], ibr.at[sl], sem))
        for d in range(D):
            copies.append(
                pltpu.async_copy(uemb_t.at[d].at[uid_v.at[j]], ur_t.at[d].at[sl], sem))
            copies.append(
                pltpu.async_copy(iemb_t.at[d].at[iid_v.at[j]], ir_t.at[d].at[sl], sem))
    for c in copies:
        c.wait()

    gb = gbv[...]

    def body(blk, carry):
        bsl = pl.ds(blk * 16, 16)
        acc = ubr[bsl] + ibr[bsl] + gb
        for d in range(D):
            acc = acc + ur_t[d, bsl] * ir_t[d, bsl]
        outv[bsl] = acc
        return carry

    lax.fori_loop(0, BPW // 16, body, 0)
    pltpu.sync_copy(outv, out_hbm.at[pl.ds(base, BPW)])


def kernel(user_id, item_id, user_emb, item_emb, user_bias, item_bias, global_bias):
    uid = user_id.astype(jnp.int32).reshape(NW * NCHUNK, CHUNK)
    iid = item_id.astype(jnp.int32).reshape(NW * NCHUNK, CHUNK)
    gb = jnp.broadcast_to(global_bias.astype(jnp.float32), (16,))
    return _mf_sc(uid, iid, user_emb.T, item_emb.T, user_bias.T, item_bias.T, gb)


# speedup vs baseline: 5.9662x; 5.9662x over previous
import functools

import numpy as np
import jax
import jax.numpy as jnp
from jax import lax
from jax.experimental import pallas as pl
from jax.experimental.pallas import tpu as pltpu
from jax.experimental.pallas import tpu_sc as plsc

B = 16384
D = 32
NC = 2
NS = 16
NW = NC * NS
BPW = B // NW
CHUNK = 128
NCHUNK = BPW // CHUNK


@functools.partial(
    pl.kernel,
    out_type=jax.ShapeDtypeStruct((B,), jnp.float32),
    mesh=plsc.VectorSubcoreMesh(core_axis_name="c", subcore_axis_name="s"),
    compiler_params=pltpu.CompilerParams(use_tc_tiling_on_sc=False),
    scratch_types=[
        pltpu.VMEM((NCHUNK, CHUNK), jnp.int32),
        pltpu.VMEM((NCHUNK, CHUNK), jnp.int32),
        pltpu.VMEM((BPW, D), jnp.float32),
        pltpu.VMEM((BPW, D), jnp.float32),
        pltpu.VMEM((BPW,), jnp.float32),
        pltpu.VMEM((BPW,), jnp.float32),
        pltpu.VMEM((16,), jnp.float32),
        pltpu.VMEM((BPW,), jnp.float32),
        pltpu.SemaphoreType.DMA,
    ],
)
def _mf_sc(uid_hbm, iid_hbm, uemb_hbm, iemb_hbm, ub_hbm, ib_hbm, gb_hbm,
           out_hbm, uid_v, iid_v, urows, irows, ubr, ibr, gbv, outv, sem):
    wid = lax.axis_index("s") * NC + lax.axis_index("c")
    base = wid * BPW

    for j in range(NCHUNK):
        row = wid * NCHUNK + j
        pltpu.sync_copy(uid_hbm.at[row], uid_v.at[j])
        pltpu.sync_copy(iid_hbm.at[row], iid_v.at[j])
    pltpu.sync_copy(gb_hbm, gbv)

    copies = []
    for j in range(NCHUNK):
        sl = pl.ds(j * CHUNK, CHUNK)
        copies.append(pltpu.async_copy(uemb_hbm.at[uid_v.at[j]], urows.at[sl], sem))
        copies.append(pltpu.async_copy(iemb_hbm.at[iid_v.at[j]], irows.at[sl], sem))
        copies.append(pltpu.async_copy(ub_hbm.at[0].at[uid_v.at[j]], ubr.at[sl], sem))
        copies.append(pltpu.async_copy(ib_hbm.at[0].at[iid_v.at[j]], ibr.at[sl], sem))
    for c in copies:
        c.wait()

    gb = gbv[...]

    def take16(v, idx):
        return lax.gather(
            v, idx[:, None],
            lax.GatherDimensionNumbers(
                offset_dims=(), collapsed_slice_dims=(0,), start_index_map=(0,)),
            slice_sizes=(1,),
            mode=lax.GatherScatterMode.PROMISE_IN_BOUNDS)

    lane = lax.iota(jnp.int32, 16)
    perms = [lane ^ k for k in (1, 2, 4, 8)]

    def body(blk, carry):
        base16 = blk * 16
        acc = jnp.zeros((16,), jnp.float32)
        for r in range(16):
            row = base16 + r
            u0 = urows[row, pl.ds(0, 16)]
            u1 = urows[row, pl.ds(16, 16)]
            i0 = irows[row, pl.ds(0, 16)]
            i1 = irows[row, pl.ds(16, 16)]
            v = u0 * i0 + u1 * i1
            for p in perms:
                v = v + take16(v, p)
            acc = jnp.where(lane == r, v, acc)
        bsl = pl.ds(base16, 16)
        outv[bsl] = acc + ubr[bsl] + ibr[bsl] + gb
        return carry

    lax.fori_loop(0, BPW // 16, body, 0)
    pltpu.sync_copy(outv, out_hbm.at[pl.ds(base, BPW)])


def kernel(user_id, item_id, user_emb, item_emb, user_bias, item_bias, global_bias):
    uid = user_id.astype(jnp.int32).reshape(NW * NCHUNK, CHUNK)
    iid = item_id.astype(jnp.int32).reshape(NW * NCHUNK, CHUNK)
    gb = jnp.broadcast_to(global_bias.astype(jnp.float32), (16,))
    return _mf_sc(uid, iid, user_emb, item_emb, user_bias.T, item_bias.T, gb)
